# baseline (speedup 1.0000x reference)
import jax
import jax.numpy as jnp
from jax import lax
from jax.experimental import pallas as pl
from jax.experimental.pallas import tpu as pltpu
from jax.experimental.pallas import tpu_sc as plsc

_M = 7
_N = 16
_A = 8192
_TK = 250
_NW = 32
_AS = _A // _NW
_AH = _AS // 2


def _sc_body(le_tab, idx_a_hbm, idx_b_hbm, mult_hbm, out_hbm,
             idx_a_v, idx_b_v, mult_v, a_bufs, b_bufs, o_bufs,
             sem_ga, sem_gb, sem_o0, sem_o1):
    wid = lax.axis_index("s") * 2 + lax.axis_index("c")
    pltpu.sync_copy(idx_a_hbm.at[wid], idx_a_v)
    pltpu.sync_copy(idx_b_hbm.at[wid], idx_b_v)
    pltpu.sync_copy(mult_hbm, mult_v)
    a0 = wid * _AS
    sem_o = (sem_o0, sem_o1)

    def issue_gather(tk, slot):
        idxa = idx_a_v[pl.ds(tk * _N, _N)]
        idxb = idx_b_v[pl.ds(tk * _N, _N)]
        pltpu.async_copy(le_tab.at[idxa], a_bufs.at[slot], sem_ga)
        pltpu.async_copy(le_tab.at[idxb], b_bufs.at[slot], sem_gb)

    def wait_gather(slot):
        pltpu.make_async_copy(le_tab.at[pl.ds(0, _N)], a_bufs.at[slot], sem_ga).wait()
        pltpu.make_async_copy(le_tab.at[pl.ds(0, _N)], b_bufs.at[slot], sem_gb).wait()

    def out_slice(tk, h):
        return out_hbm.at[tk, pl.ds(h * (_N // 2), _N // 2), :, pl.ds(a0, _AS)]

    def wait_scatter(h, tk):
        pltpu.make_async_copy(o_bufs.at[h], out_slice(tk, h), sem_o[h]).wait()

    def compute_half(slot, h, mv):
        i0 = h * (_N // 2)

        def c_body(c, _):
            cs = pl.ds(c * 16, 16)
            a_regs = [a_bufs[slot, i0 + i, cs] for i in range(_N // 2)]
            for j in range(_N):
                bmj = b_bufs[slot, j, cs] * mv
                for i in range(_N // 2):
                    o_bufs[h, i, j, cs] = a_regs[i] * bmj
            return 0

        lax.fori_loop(0, _AS // 16, c_body, 0, unroll=2)

    issue_gather(0, 0)

    @pl.loop(0, _TK, step=2)
    def pair(tk0):
        for s in range(2):
            tk = tk0 + s
            wait_gather(slot=s)
            if s == 0:
                issue_gather(tk0 + 1, 1)
            else:
                @pl.when(tk0 + 2 < _TK)
                def _():
                    issue_gather(tk0 + 2, 0)
            mv = mult_v[tk]
            for h in range(2):
                if s == 0:
                    @pl.when(tk0 > 0)
                    def _():
                        wait_scatter(h, tk)
                else:
                    wait_scatter(h, tk)
                compute_half(s, h, mv)
                pltpu.async_copy(o_bufs.at[h], out_slice(tk, h), sem_o[h])

    wait_scatter(0, _TK - 1)
    wait_scatter(1, _TK - 1)


def kernel(LE_1, indices_prev, indices_1, l_tuples, multiplicities_t):
    T, K = indices_prev.shape
    rows_a = (l_tuples[:, 0][:, None] * _M + indices_prev).reshape(-1)
    rows_b = (l_tuples[:, 1][:, None] * _M + indices_1).reshape(-1)
    rows_a = rows_a.astype(jnp.int32)
    rows_b = rows_b.astype(jnp.int32)
    n_off = jnp.arange(_N, dtype=jnp.int32)[None, :] * _NW
    w_off = jnp.arange(_NW, dtype=jnp.int32)[:, None]
    idx_a = (rows_a[:, None] * (_N * _NW) + n_off).reshape(1, -1) + w_off
    idx_b = (rows_b[:, None] * (_N * _NW) + n_off).reshape(1, -1) + w_off
    mult_b = jnp.broadcast_to(
        multiplicities_t.reshape(-1)[:, None], (_TK, _N)
    ).astype(jnp.float32)
    le_tab = LE_1.reshape(-1, _AS)

    mesh = plsc.VectorSubcoreMesh(core_axis_name="c", subcore_axis_name="s")
    sck = pl.kernel(
        _sc_body,
        out_type=jax.ShapeDtypeStruct((_TK, _N, _N, _A), jnp.float32),
        mesh=mesh,
        scratch_types=[
            pltpu.VMEM((_TK * _N,), jnp.int32),
            pltpu.VMEM((_TK * _N,), jnp.int32),
            pltpu.VMEM((_TK, _N), jnp.float32),
            pltpu.VMEM((2, _N, _AS), jnp.float32),
            pltpu.VMEM((2, _N, _AS), jnp.float32),
            pltpu.VMEM((2, _N // 2, _N, _AS), jnp.float32),
            pltpu.SemaphoreType.DMA,
            pltpu.SemaphoreType.DMA,
            pltpu.SemaphoreType.DMA,
            pltpu.SemaphoreType.DMA,
        ],
    )
    out = sck(le_tab, idx_a, idx_b, mult_b)
    return out.reshape(T, K, _N * _N, _A)


# speedup vs baseline: 1.8507x; 1.8507x over previous
import jax
import jax.numpy as jnp
from jax import lax
from jax.experimental import pallas as pl
from jax.experimental.pallas import tpu as pltpu
from jax.experimental.pallas import tpu_sc as plsc

_M = 7
_N = 16
_A = 8192
_TK = 250
_NW = 32
_AS = _A // _NW
_AH = _AS // 2


def _sc_body(le_tab, idx_a_hbm, idx_b_hbm, mult_hbm, out_hbm,
             idx_a_v, idx_b_v, mult_v, a_bufs, b_bufs, o_bufs,
             sem_ga, sem_gb, sem_o0, sem_o1):
    wid = lax.axis_index("s") * 2 + lax.axis_index("c")
    pltpu.sync_copy(idx_a_hbm.at[wid], idx_a_v)
    pltpu.sync_copy(idx_b_hbm.at[wid], idx_b_v)
    pltpu.sync_copy(mult_hbm, mult_v)
    a0 = wid * _AS
    sem_o = (sem_o0, sem_o1)

    def issue_gather(tk, slot):
        idxa = idx_a_v[pl.ds(tk * _N, _N)]
        idxb = idx_b_v[pl.ds(tk * _N, _N)]
        pltpu.async_copy(le_tab.at[idxa], a_bufs.at[slot], sem_ga)
        pltpu.async_copy(le_tab.at[idxb], b_bufs.at[slot], sem_gb)

    def wait_gather(slot):
        pltpu.make_async_copy(le_tab.at[pl.ds(0, _N)], a_bufs.at[slot], sem_ga).wait()
        pltpu.make_async_copy(le_tab.at[pl.ds(0, _N)], b_bufs.at[slot], sem_gb).wait()

    def out_slice(tk, h):
        return out_hbm.at[tk, pl.ds(h * (_N // 2), _N // 2), :, pl.ds(a0, _AS)]

    def wait_scatter(h, tk):
        pltpu.make_async_copy(o_bufs.at[h], out_slice(tk, h), sem_o[h]).wait()

    def compute_half(slot, h, mv):
        i0 = h * (_N // 2)

        def c_body(c, _):
            cs = pl.ds(c * 16, 16)
            a_regs = [a_bufs[slot, i0 + i, cs] for i in range(_N // 2)]
            for j in range(_N):
                bmj = b_bufs[slot, j, cs] * mv
                for i in range(_N // 2):
                    o_bufs[h, i, j, cs] = a_regs[i] * bmj
            return 0

        lax.fori_loop(0, _AS // 16, c_body, 0)

    issue_gather(0, 0)

    @pl.loop(0, _TK, step=2)
    def pair(tk0):
        for s in range(2):
            tk = tk0 + s
            wait_gather(slot=s)
            if s == 0:
                issue_gather(tk0 + 1, 1)
            else:
                @pl.when(tk0 + 2 < _TK)
                def _():
                    issue_gather(tk0 + 2, 0)
            mv = mult_v[tk]
            for h in range(2):
                if s == 0:
                    @pl.when(tk0 > 0)
                    def _():
                        wait_scatter(h, tk)
                else:
                    wait_scatter(h, tk)
                compute_half(s, h, mv)
                pltpu.async_copy(o_bufs.at[h], out_slice(tk, h), sem_o[h])

    wait_scatter(0, _TK - 1)
    wait_scatter(1, _TK - 1)


def kernel(LE_1, indices_prev, indices_1, l_tuples, multiplicities_t):
    T, K = indices_prev.shape
    rows_a = (l_tuples[:, 0][:, None] * _M + indices_prev).reshape(-1)
    rows_b = (l_tuples[:, 1][:, None] * _M + indices_1).reshape(-1)
    rows_a = rows_a.astype(jnp.int32)
    rows_b = rows_b.astype(jnp.int32)
    n_off = jnp.arange(_N, dtype=jnp.int32)[None, :] * _NW
    w_off = jnp.arange(_NW, dtype=jnp.int32)[:, None]
    idx_a = (rows_a[:, None] * (_N * _NW) + n_off).reshape(1, -1) + w_off
    idx_b = (rows_b[:, None] * (_N * _NW) + n_off).reshape(1, -1) + w_off
    mult_b = jnp.broadcast_to(
        multiplicities_t.reshape(-1)[:, None], (_TK, _N)
    ).astype(jnp.float32)
    le_tab = LE_1.reshape(-1, _AS)

    mesh = plsc.VectorSubcoreMesh(core_axis_name="c", subcore_axis_name="s")
    sck = pl.kernel(
        _sc_body,
        out_type=jax.ShapeDtypeStruct((_TK, _N, _N, _A), jnp.float32),
        mesh=mesh,
        scratch_types=[
            pltpu.VMEM((_TK * _N,), jnp.int32),
            pltpu.VMEM((_TK * _N,), jnp.int32),
            pltpu.VMEM((_TK, _N), jnp.float32),
            pltpu.VMEM((2, _N, _AS), jnp.float32),
            pltpu.VMEM((2, _N, _AS), jnp.float32),
            pltpu.VMEM((2, _N // 2, _N, _AS), jnp.float32),
            pltpu.SemaphoreType.DMA,
            pltpu.SemaphoreType.DMA,
            pltpu.SemaphoreType.DMA,
            pltpu.SemaphoreType.DMA,
        ],
    )
    out = sck(le_tab, idx_a, idx_b, mult_b)
    return out.reshape(T, K, _N * _N, _A)
